# baseline (speedup 1.0000x reference)
import functools

import jax
import jax.numpy as jnp
from jax import lax
from jax.experimental import pallas as pl
from jax.experimental.pallas import tpu as pltpu
from jax.experimental.pallas import tpu_sc as plsc

N_CLS = 4096
N_CTX = 16
MAX_NAME = 16
D = 768
MAX_LEN = 1 + N_CTX + MAX_NAME + 1
HEAD = 1 + N_CTX
TAIL = MAX_NAME + 1

NC = 2
NS = 16
NW = NC * NS
PER_TILE = N_CLS // NW
NBUF = 4
STEPS = PER_TILE // NBUF

SIZES = tuple(range(2, TAIL + 1))


def _sized(n, fn):
    prev = 0
    for s in SIZES:
        @pl.when(jnp.logical_and(n > prev, n <= s))
        def _(s=s):
            fn(s)
        prev = s


def _sc_body(table_hbm, ct_hbm, lens_hbm, par_hbm,
             o_ref,
             stag0, stag1, stag2, stag3,
             gidx0, gidx1, gidx2, gidx3,
             ct_v, lens_v, par_v,
             gsem0, gsem1, gsem2, gsem3,
             osem0, osem1, osem2, osem3):
    stags = (stag0, stag1, stag2, stag3)
    gidxs = (gidx0, gidx1, gidx2, gidx3)
    gsems = (gsem0, gsem1, gsem2, gsem3)
    osems = (osem0, osem1, osem2, osem3)

    wid = lax.axis_index("s") * NC + lax.axis_index("c")
    base = wid * PER_TILE
    iota = lax.broadcasted_iota(jnp.int32, (16,), 0)

    pltpu.sync_copy(par_hbm, par_v)
    pltpu.sync_copy(ct_hbm.at[pl.ds(base, PER_TILE)], ct_v)
    pltpu.sync_copy(lens_hbm.at[pl.ds(base, PER_TILE)], lens_v)

    def n_rows(g):
        lenv = plsc.load_gather(lens_v, [jnp.full((16,), g, jnp.int32)])
        return jnp.max(lenv) + 1

    sep_v = plsc.load_gather(par_v, [iota * 0 + 1])

    def fill_idx(b, g):
        lsp = jnp.full((16,), g, jnp.int32)
        tok = plsc.load_gather(ct_v, [lsp, iota])
        lenv = plsc.load_gather(lens_v, [lsp])
        idx16 = jnp.where(iota < lenv, tok, sep_v)
        plsc.store_scatter(gidxs[b], [iota], idx16)
        plsc.store_scatter(gidxs[b], [iota * 0 + 16], sep_v, mask=iota == 0)

    def dst(g, s):
        return o_ref.at[pl.ds((base + g) * MAX_LEN + HEAD, s)]

    def issue_gather(b, g):
        def go(s):
            pltpu.async_copy(table_hbm.at[gidxs[b].at[pl.ds(0, s)]],
                             stags[b].at[pl.ds(0, s)], gsems[b])
        _sized(n_rows(g), go)

    def wait_gather_issue_out(b, g):
        def go(s):
            pltpu.make_async_copy(table_hbm.at[gidxs[b].at[pl.ds(0, s)]],
                                  stags[b].at[pl.ds(0, s)], gsems[b]).wait()
            pltpu.async_copy(stags[b].at[pl.ds(0, s)], dst(g, s), osems[b])
        _sized(n_rows(g), go)

    def wait_out(b, g):
        def go(s):
            pltpu.make_async_copy(
                stags[b].at[pl.ds(0, s)], dst(g, s), osems[b]).wait()
        _sized(n_rows(g), go)

    def step(st, carry):
        for b in range(NBUF):
            g = st * NBUF + b

            @pl.when(g >= NBUF)
            def _():
                wait_out(b, g - NBUF)

            fill_idx(b, g)
            issue_gather(b, g)
        for b in range(NBUF):
            wait_gather_issue_out(b, st * NBUF + b)
        return carry

    lax.fori_loop(0, STEPS, step, 0)
    for b in range(NBUF):
        wait_out(b, (STEPS - 1) * NBUF + b)


def _sc_scatter_into(table, class_tokens, lens, par, o_ref):
    mesh = plsc.VectorSubcoreMesh(core_axis_name="c", subcore_axis_name="s")
    f = pl.kernel(
        _sc_body,
        mesh=mesh,
        compiler_params=pltpu.CompilerParams(use_tc_tiling_on_sc=False,
                                             needs_layout_passes=False),
        out_type=(),
        scratch_types=(
            [pltpu.VMEM((TAIL, D), jnp.float32)] * NBUF
            + [pltpu.VMEM((TAIL,), jnp.int32)] * NBUF
            + [
                pltpu.VMEM((PER_TILE, MAX_NAME), jnp.int32),
                pltpu.VMEM((PER_TILE,), jnp.int32),
                pltpu.VMEM((8,), jnp.int32),
            ]
            + [pltpu.SemaphoreType.DMA] * (2 * NBUF)
        ),
    )
    f(table, class_tokens, lens, par, o_ref)


BC = 64


def _tc_fill(base_ref, lens_ref, out_ref, mask_ref):
    lenb = lens_ref[...]
    head = jnp.broadcast_to(base_ref[...][None], (BC, HEAD, D))
    tail = jnp.zeros((BC, TAIL, D), jnp.float32)
    out_ref[...] = jnp.concatenate([head, tail], axis=1)
    p_iota = lax.broadcasted_iota(jnp.int32, (BC, MAX_LEN), 1)
    mask_ref[...] = (p_iota < 18 + lenb).astype(jnp.int32)


def _tc_prefill(base, lens2):
    return pl.pallas_call(
        _tc_fill,
        grid=(N_CLS // BC,),
        in_specs=[
            pl.BlockSpec((HEAD, D), lambda i: (0, 0)),
            pl.BlockSpec((BC, 1), lambda i: (i, 0)),
        ],
        out_specs=[
            pl.BlockSpec((BC, MAX_LEN, D), lambda i: (i, 0, 0)),
            pl.BlockSpec((BC, MAX_LEN), lambda i: (i, 0)),
        ],
        out_shape=[
            jax.ShapeDtypeStruct((N_CLS, MAX_LEN, D), jnp.float32),
            jax.ShapeDtypeStruct((N_CLS, MAX_LEN), jnp.int32),
        ],
    )(base, lens2)


def kernel(table, ctx, class_tokens, lens, cls_id, sep_id):
    par = (jnp.zeros((8,), jnp.int32)
           .at[0].set(jnp.asarray(cls_id, jnp.int32))
           .at[1].set(jnp.asarray(sep_id, jnp.int32)))
    base = jnp.concatenate([table[cls_id][None, :], ctx], axis=0)
    out, out_mask = _tc_prefill(base, lens[:, None])
    o_ref = jax.new_ref(out.reshape(N_CLS * MAX_LEN, D))
    _sc_scatter_into(table, class_tokens, lens, par, o_ref)
    return o_ref[...].reshape(N_CLS, MAX_LEN, D), out_mask


# speedup vs baseline: 1.4612x; 1.4612x over previous
import functools

import jax
import jax.numpy as jnp
from jax import lax
from jax.experimental import pallas as pl
from jax.experimental.pallas import tpu as pltpu
from jax.experimental.pallas import tpu_sc as plsc

N_CLS = 4096
N_CTX = 16
MAX_NAME = 16
D = 768
MAX_LEN = 1 + N_CTX + MAX_NAME + 1
HEAD = 1 + N_CTX
TAIL = MAX_NAME + 1

NC = 2
NS = 16
NW = NC * NS
HALF = N_CLS // 2
PER_TILE = HALF // NW
NBUF = 4
STEPS = PER_TILE // NBUF

SIZES = tuple(range(2, TAIL + 1))


def _sized(n, fn):
    prev = 0
    for s in SIZES:
        @pl.when(jnp.logical_and(n > prev, n <= s))
        def _(s=s):
            fn(s)
        prev = s


def _sc_body(table_hbm, ct_hbm, lens_hbm, par_hbm,
             t_hbm,
             stag0, stag1, stag2, stag3,
             gidx0, gidx1, gidx2, gidx3,
             ct_v, lens_v, par_v,
             gsem0, gsem1, gsem2, gsem3,
             osem0, osem1, osem2, osem3):
    stags = (stag0, stag1, stag2, stag3)
    gidxs = (gidx0, gidx1, gidx2, gidx3)
    gsems = (gsem0, gsem1, gsem2, gsem3)
    osems = (osem0, osem1, osem2, osem3)

    wid = lax.axis_index("s") * NC + lax.axis_index("c")
    base = wid * PER_TILE
    iota = lax.broadcasted_iota(jnp.int32, (16,), 0)

    pltpu.sync_copy(par_hbm, par_v)
    pltpu.sync_copy(ct_hbm.at[pl.ds(base, PER_TILE)], ct_v)
    pltpu.sync_copy(lens_hbm.at[pl.ds(base, PER_TILE)], lens_v)

    def n_rows(g):
        lenv = plsc.load_gather(lens_v, [jnp.full((16,), g, jnp.int32)])
        return jnp.max(lenv) + 1
    sep_v = plsc.load_gather(par_v, [iota * 0 + 1])

    def fill_idx(b, g):
        lsp = jnp.full((16,), g, jnp.int32)
        tok = plsc.load_gather(ct_v, [lsp, iota])
        lenv = plsc.load_gather(lens_v, [lsp])
        idx16 = jnp.where(iota < lenv, tok, sep_v)
        plsc.store_scatter(gidxs[b], [iota], idx16)
        plsc.store_scatter(gidxs[b], [iota * 0 + 16], sep_v, mask=iota == 0)

    def issue_gather(b, g):
        def go(s):
            pltpu.async_copy(table_hbm.at[gidxs[b].at[pl.ds(0, s)]],
                             stags[b].at[pl.ds(0, s)], gsems[b])
        _sized(n_rows(g), go)

    def wait_gather_issue_out(b, g):
        def go(s):
            pltpu.make_async_copy(table_hbm.at[gidxs[b].at[pl.ds(0, s)]],
                                  stags[b].at[pl.ds(0, s)], gsems[b]).wait()
            pltpu.async_copy(stags[b].at[pl.ds(0, s)],
                             t_hbm.at[pl.ds((base + g) * TAIL, s)], osems[b])
        _sized(n_rows(g), go)

    def wait_out(b, g):
        def go(s):
            pltpu.make_async_copy(
                stags[b].at[pl.ds(0, s)],
                t_hbm.at[pl.ds((base + g) * TAIL, s)], osems[b]).wait()
        _sized(n_rows(g), go)

    def step(st, carry):
        for b in range(NBUF):
            g = st * NBUF + b

            @pl.when(g >= NBUF)
            def _():
                wait_out(b, g - NBUF)

            fill_idx(b, g)
            issue_gather(b, g)
        for b in range(NBUF):
            wait_gather_issue_out(b, st * NBUF + b)
        return carry

    lax.fori_loop(0, STEPS, step, 0)
    for b in range(NBUF):
        wait_out(b, (STEPS - 1) * NBUF + b)


def _sc_gather(table, class_tokens, lens, par):
    mesh = plsc.VectorSubcoreMesh(core_axis_name="c", subcore_axis_name="s")
    f = pl.kernel(
        _sc_body,
        mesh=mesh,
        compiler_params=pltpu.CompilerParams(use_tc_tiling_on_sc=False,
                                             needs_layout_passes=False),
        out_type=jax.ShapeDtypeStruct((HALF * TAIL, D), jnp.float32),
        scratch_types=(
            [pltpu.VMEM((TAIL, D), jnp.float32)] * NBUF
            + [pltpu.VMEM((TAIL,), jnp.int32)] * NBUF
            + [
                pltpu.VMEM((PER_TILE, MAX_NAME), jnp.int32),
                pltpu.VMEM((PER_TILE,), jnp.int32),
                pltpu.VMEM((8,), jnp.int32),
            ]
            + [pltpu.SemaphoreType.DMA] * (2 * NBUF)
        ),
    )
    return f(table, class_tokens, lens, par)


BC = 64


def _tc_compute(t_ref, base_ref, lens_ref, out_ref, mask_ref):
    lenb = lens_ref[...]
    s_iota = lax.broadcasted_iota(jnp.int32, (BC, TAIL, 1), 1)
    tail = jnp.where(s_iota <= lenb[:, :, None], t_ref[...], 0.0)
    head = jnp.broadcast_to(base_ref[...][None], (BC, HEAD, D))
    out_ref[...] = jnp.concatenate([head, tail], axis=1)
    p_iota = lax.broadcasted_iota(jnp.int32, (BC, MAX_LEN), 1)
    mask_ref[...] = (p_iota < 18 + lenb).astype(jnp.int32)


_OUT_SHAPES = [
    jax.ShapeDtypeStruct((N_CLS, MAX_LEN, D), jnp.float32),
    jax.ShapeDtypeStruct((N_CLS, MAX_LEN), jnp.int32),
]
_OUT_SPECS = [
    pl.BlockSpec((BC, MAX_LEN, D), lambda i: (i, 0, 0)),
    pl.BlockSpec((BC, MAX_LEN), lambda i: (i, 0)),
]
_OUT_SPECS_HI = [
    pl.BlockSpec((BC, MAX_LEN, D), lambda i: (i + HALF // BC, 0, 0)),
    pl.BlockSpec((BC, MAX_LEN), lambda i: (i + HALF // BC, 0)),
]
_IN_SPECS = [
    pl.BlockSpec((BC, TAIL, D), lambda i: (i, 0, 0)),
    pl.BlockSpec((HEAD, D), lambda i: (0, 0)),
    pl.BlockSpec((BC, 1), lambda i: (i, 0)),
]


def _tc_assemble_lo(t, base, lens2):
    return pl.pallas_call(
        _tc_compute,
        grid=(HALF // BC,),
        in_specs=_IN_SPECS,
        out_specs=_OUT_SPECS,
        out_shape=_OUT_SHAPES,
    )(t, base, lens2)


def _tc_body_hi(t_ref, base_ref, lens_ref, _prev_out, _prev_mask,
                out_ref, mask_ref):
    _tc_compute(t_ref, base_ref, lens_ref, out_ref, mask_ref)


def _tc_assemble_hi(t, base, lens2, prev_out, prev_mask):
    return pl.pallas_call(
        _tc_body_hi,
        grid=(HALF // BC,),
        in_specs=_IN_SPECS + [
            pl.BlockSpec(memory_space=pltpu.MemorySpace.HBM),
            pl.BlockSpec(memory_space=pltpu.MemorySpace.HBM),
        ],
        out_specs=_OUT_SPECS_HI,
        out_shape=_OUT_SHAPES,
        input_output_aliases={3: 0, 4: 1},
    )(t, base, lens2, prev_out, prev_mask)


def kernel(table, ctx, class_tokens, lens, cls_id, sep_id):
    par = (jnp.zeros((8,), jnp.int32)
           .at[0].set(jnp.asarray(cls_id, jnp.int32))
           .at[1].set(jnp.asarray(sep_id, jnp.int32)))
    base = jnp.concatenate([table[cls_id][None, :], ctx], axis=0)
    t0 = _sc_gather(table, class_tokens[:HALF], lens[:HALF], par)
    t1 = _sc_gather(table, class_tokens[HALF:], lens[HALF:], par)
    out_a, mask_a = _tc_assemble_lo(
        t0.reshape(HALF, TAIL, D), base, lens[:HALF, None])
    out_embeds, out_mask = _tc_assemble_hi(
        t1.reshape(HALF, TAIL, D), base, lens[HALF:, None], out_a, mask_a)
    return out_embeds, out_mask
